# baseline (speedup 1.0000x reference)
import functools

import jax
import jax.numpy as jnp
from jax import lax
from jax.experimental import pallas as pl
from jax.experimental.pallas import tpu as pltpu
from jax.experimental.pallas import tpu_sc as plsc

_N = 10000
_E = 320000
_D = 128
_H = 64
_DE = 16
_L = 3
_G = 256

_NP = 10240
_B = 64
_EP = 323584
_PER_S = _EP // 16
_NBATCH = _PER_S // _B
_STRIPE = _NP // 16


def _edge_embed_body(ef_ref, w_ref, b_ref, e0_ref, e1_ref, e2_ref):
    y = jnp.dot(ef_ref[...], w_ref[...], preferred_element_type=jnp.float32)
    y = y + b_ref[...]
    for l, ref in enumerate((e0_ref, e1_ref, e2_ref)):
        ref[0] = y[:, l * _D:l * _D + _H]
        ref[1] = y[:, l * _D + _H:(l + 1) * _D]


def _edge_embed(ef, w, b):
    be = 2048
    eshape = jax.ShapeDtypeStruct((2, _EP, _H), jnp.float32)
    espec = pl.BlockSpec((2, be, _H), lambda i: (0, i, 0))
    return pl.pallas_call(
        _edge_embed_body,
        grid=(_EP // be,),
        in_specs=[
            pl.BlockSpec((be, _DE), lambda i: (i, 0)),
            pl.BlockSpec((_DE, _L * _D), lambda i: (0, 0)),
            pl.BlockSpec((1, _L * _D), lambda i: (0, 0)),
        ],
        out_specs=[espec, espec, espec],
        out_shape=[eshape, eshape, eshape],
        compiler_params=pltpu.CompilerParams(
            dimension_semantics=("parallel",)),
    )(ef, w, b)


def _split_outs(y, k_ref, qv_ref, xs_ref):
    k_ref[0] = y[:, 0:_H]
    k_ref[1] = y[:, _H:_D]
    qv_ref[0] = jnp.concatenate(
        [y[:, _D:_D + _H], y[:, 2 * _D:2 * _D + _H]], axis=1)
    qv_ref[1] = jnp.concatenate(
        [y[:, _D + _H:2 * _D], y[:, 2 * _D + _H:3 * _D]], axis=1)
    xs_ref[...] = y[:, 3 * _D:]


def _dense0_body(x_ref, w_ref, b_ref, k_ref, qv_ref, xs_ref):
    y = jnp.dot(x_ref[...], w_ref[...], preferred_element_type=jnp.float32)
    _split_outs(y + b_ref[...], k_ref, qv_ref, xs_ref)


def _dense_relu_body(p0_ref, p1_ref, xsin_ref, w_ref, b_ref,
                     k_ref, qv_ref, xs_ref):
    agg = jnp.concatenate([p0_ref[0], p1_ref[0]], axis=1)
    x = jnp.maximum(agg + xsin_ref[...], 0.0)
    y = jnp.dot(x, w_ref[...], preferred_element_type=jnp.float32)
    _split_outs(y + b_ref[...], k_ref, qv_ref, xs_ref)


def _dense_outs():
    return [
        jax.ShapeDtypeStruct((2, _NP, _H), jnp.float32),
        jax.ShapeDtypeStruct((2, _NP, 2 * _H), jnp.float32),
        jax.ShapeDtypeStruct((_NP, _D), jnp.float32),
    ]


def _dense_out_specs(bn):
    return [
        pl.BlockSpec((2, bn, _H), lambda i: (0, i, 0)),
        pl.BlockSpec((2, bn, 2 * _H), lambda i: (0, i, 0)),
        pl.BlockSpec((bn, _D), lambda i: (i, 0)),
    ]


def _dense0(x, w, b):
    bn = 512
    return pl.pallas_call(
        _dense0_body,
        grid=(_NP // bn,),
        in_specs=[
            pl.BlockSpec((bn, _D), lambda i: (i, 0)),
            pl.BlockSpec((_D, 4 * _D), lambda i: (0, 0)),
            pl.BlockSpec((1, 4 * _D), lambda i: (0, 0)),
        ],
        out_specs=_dense_out_specs(bn),
        out_shape=_dense_outs(),
        compiler_params=pltpu.CompilerParams(
            dimension_semantics=("parallel",)),
    )(x, w, b)


def _dense_relu(parts, xs_in, w, b):
    bn = 512
    return pl.pallas_call(
        _dense_relu_body,
        grid=(_NP // bn,),
        in_specs=[
            pl.BlockSpec((1, bn, _H), lambda i: (0, i, 0)),
            pl.BlockSpec((1, bn, _H), lambda i: (1, i, 0)),
            pl.BlockSpec((bn, _D), lambda i: (i, 0)),
            pl.BlockSpec((_D, 4 * _D), lambda i: (0, 0)),
            pl.BlockSpec((1, 4 * _D), lambda i: (0, 0)),
        ],
        out_specs=_dense_out_specs(bn),
        out_shape=_dense_outs(),
        compiler_params=pltpu.CompilerParams(
            dimension_semantics=("parallel",)),
    )(parts, parts, xs_in, w, b)


def _sc_edge(k, qv, e, src, dst, zeros):
    mesh = plsc.VectorSubcoreMesh(core_axis_name="c", subcore_axis_name="s")

    @functools.partial(
        pl.kernel,
        out_type=jax.ShapeDtypeStruct((2, _NP, _H), jnp.float32),
        mesh=mesh,
        scratch_types=[
            pltpu.VMEM((_NBATCH, _B), jnp.int32),
            pltpu.VMEM((_NBATCH, _B), jnp.int32),
            [pltpu.VMEM((_B, _H), jnp.float32) for _ in range(2)],
            [pltpu.VMEM((_B, 2 * _H), jnp.float32) for _ in range(2)],
            [pltpu.VMEM((_B, _H), jnp.float32) for _ in range(2)],
            [pltpu.VMEM((_B, _H), jnp.float32) for _ in range(2)],
            [pltpu.SemaphoreType.DMA for _ in range(2)],
            pltpu.VMEM_SHARED((_NP, _H), jnp.float32),
        ],
        compiler_params=pltpu.CompilerParams(use_tc_tiling_on_sc=False),
    )
    def kern(k_hbm, qv_hbm, e_hbm, src_hbm, dst_hbm, z_hbm, out_hbm,
             src_v, dst_v, kb, qvb, eb, mb, sem, agg):
        c = lax.axis_index("c")
        s = lax.axis_index("s")
        pltpu.sync_copy(z_hbm.at[pl.ds(s * _STRIPE, _STRIPE)],
                        agg.at[pl.ds(s * _STRIPE, _STRIPE)])
        pltpu.sync_copy(src_hbm.at[s], src_v)
        pltpu.sync_copy(dst_hbm.at[s], dst_v)
        plsc.subcore_barrier()
        base0 = s * _PER_S

        def gather_descr(bi, slot):
            base = base0 + bi * _B
            return (
                pltpu.make_async_copy(k_hbm.at[c].at[dst_v.at[bi]], kb[slot],
                                      sem[slot]),
                pltpu.make_async_copy(qv_hbm.at[c].at[src_v.at[bi]],
                                      qvb[slot], sem[slot]),
                pltpu.make_async_copy(e_hbm.at[c, pl.ds(base, _B)], eb[slot],
                                      sem[slot]),
            )

        def issue(bi, slot):
            for d in gather_descr(bi, slot):
                d.start()

        def finish(bi, slot):
            for d in gather_descr(bi, slot):
                d.wait()

            @plsc.parallel_loop(0, _B, unroll=4)
            def _row(i):
                for j in range(_H // 16):
                    sl = pl.ds(j * 16, 16)
                    z = kb[slot][i, sl] + qvb[slot][i, sl] + eb[slot][i, sl]
                    vv = qvb[slot][i, pl.ds(_H + j * 16, 16)]
                    mb[slot][i, sl] = vv / (1.0 + jnp.exp(-z))

            pltpu.sync_copy(mb[slot], agg.at[dst_v.at[bi]], add=True)

        issue(0, 0)

        @pl.loop(0, _NBATCH, step=2)
        def _batch(bi):
            issue(bi + 1, 1)
            finish(bi, 0)

            @pl.when(bi + 2 < _NBATCH)
            def _():
                issue(bi + 2, 0)

            finish(bi + 1, 1)

        plsc.subcore_barrier()
        pltpu.sync_copy(agg.at[pl.ds(s * _STRIPE, _STRIPE)],
                        out_hbm.at[c, pl.ds(s * _STRIPE, _STRIPE)])

    return kern(k, qv, e, src, dst, zeros)


def _head_body(p0_ref, p1_ref, xs_ref, b3_ref, wg_ref, bg_ref,
               lng_ref, lnb_ref, lgg_ref, lgb_ref,
               loc_ref, glob_ref, gsum_ref, gcnt_ref):
    i = pl.program_id(0)
    nb = pl.num_programs(0)
    agg = jnp.concatenate([p0_ref[0], p1_ref[0]], axis=1)
    x = agg + xs_ref[...]

    m = jnp.mean(x, axis=-1, keepdims=True)
    v = jnp.mean((x - m) ** 2, axis=-1, keepdims=True)
    loc_ref[...] = ((x - m) * lax.rsqrt(v + 1e-5) * lng_ref[...]
                    + lnb_ref[...])

    @pl.when(i == 0)
    def _():
        gsum_ref[...] = jnp.zeros_like(gsum_ref)
        gcnt_ref[...] = jnp.zeros_like(gcnt_ref)

    seg = b3_ref[0]
    segb = jnp.broadcast_to(seg, (_G, seg.shape[1]))
    gids = lax.broadcasted_iota(jnp.int32, (_G, seg.shape[1]), 0)
    mt = (segb == gids).astype(jnp.float32)
    gsum_ref[...] += jnp.dot(mt, x, preferred_element_type=jnp.float32)
    gcnt_ref[...] += jnp.dot(mt, jnp.ones_like(x),
                             preferred_element_type=jnp.float32)

    @pl.when(i == nb - 1)
    def _():
        gmean = gsum_ref[...] / jnp.maximum(gcnt_ref[...], 1.0)
        gg = jnp.dot(gmean, wg_ref[...],
                     preferred_element_type=jnp.float32) + bg_ref[...]
        m2 = jnp.mean(gg, axis=-1, keepdims=True)
        v2 = jnp.mean((gg - m2) ** 2, axis=-1, keepdims=True)
        glob_ref[...] = ((gg - m2) * lax.rsqrt(v2 + 1e-5) * lgg_ref[...]
                         + lgb_ref[...])


def _head(parts, xs_in, batch3, wg, bg, lng, lnb, lgg, lgb):
    bh = 400
    return pl.pallas_call(
        _head_body,
        grid=(_N // bh,),
        in_specs=[
            pl.BlockSpec((1, bh, _H), lambda i: (0, i, 0)),
            pl.BlockSpec((1, bh, _H), lambda i: (1, i, 0)),
            pl.BlockSpec((bh, _D), lambda i: (i, 0)),
            pl.BlockSpec((1, 1, bh), lambda i: (i, 0, 0)),
            pl.BlockSpec((_D, _D), lambda i: (0, 0)),
            pl.BlockSpec((1, _D), lambda i: (0, 0)),
            pl.BlockSpec((1, _D), lambda i: (0, 0)),
            pl.BlockSpec((1, _D), lambda i: (0, 0)),
            pl.BlockSpec((1, _D), lambda i: (0, 0)),
            pl.BlockSpec((1, _D), lambda i: (0, 0)),
        ],
        out_specs=[
            pl.BlockSpec((bh, _D), lambda i: (i, 0)),
            pl.BlockSpec((_G, _D), lambda i: (0, 0)),
        ],
        out_shape=[
            jax.ShapeDtypeStruct((_N, _D), jnp.float32),
            jax.ShapeDtypeStruct((_G, _D), jnp.float32),
        ],
        scratch_shapes=[
            pltpu.VMEM((_G, _D), jnp.float32),
            pltpu.VMEM((_G, _D), jnp.float32),
        ],
        compiler_params=pltpu.CompilerParams(
            dimension_semantics=("arbitrary",)),
    )(parts, parts, xs_in, batch3, wg, bg, lng, lnb, lgg, lgb)


def kernel(node_feature, edge_index, edge_feature, batch, Wk, bk, Wq, bq,
           Wv, bv, We, be, Ws, bconv, Wg, bg, ln_node_g, ln_node_b,
           ln_graph_g, ln_graph_b):
    f32 = jnp.float32
    x0 = jnp.pad(node_feature, ((0, _NP - _N), (0, 0)))
    src = jnp.pad(edge_index[0], (0, _EP - _E)).reshape(16, _NBATCH, _B)
    dst = jnp.pad(edge_index[1], (0, _EP - _E),
                  constant_values=_N).reshape(16, _NBATCH, _B)
    ef = jnp.pad(edge_feature, ((0, _EP - _E), (0, 0)))
    batch3 = batch.reshape(_N // 400, 1, 400)
    zeros = jnp.zeros((_NP, _H), f32)

    wall = jnp.concatenate([Wk, Wq, Wv, Ws], axis=2)
    ball = jnp.concatenate(
        [bk, bq, bv, bconv], axis=1).reshape(_L, 1, 4 * _D)
    we_all = jnp.transpose(We, (1, 0, 2)).reshape(_DE, _L * _D)
    be_all = be.reshape(1, _L * _D)

    e0, e1, e2 = _edge_embed(ef, we_all, be_all)

    k, qv, xs = _dense0(x0, wall[0], ball[0])
    parts = _sc_edge(k, qv, e0, src, dst, zeros)
    k, qv, xs = _dense_relu(parts, xs, wall[1], ball[1])
    parts = _sc_edge(k, qv, e1, src, dst, zeros)
    k, qv, xs = _dense_relu(parts, xs, wall[2], ball[2])
    parts = _sc_edge(k, qv, e2, src, dst, zeros)

    loc, glob = _head(parts, xs, batch3,
                      Wg, bg.reshape(1, _D),
                      ln_node_g.reshape(1, _D), ln_node_b.reshape(1, _D),
                      ln_graph_g.reshape(1, _D), ln_graph_b.reshape(1, _D))
    return loc, glob


# speedup vs baseline: 1.0137x; 1.0137x over previous
import functools

import jax
import jax.numpy as jnp
from jax import lax
from jax.experimental import pallas as pl
from jax.experimental.pallas import tpu as pltpu
from jax.experimental.pallas import tpu_sc as plsc

_N = 10000
_E = 320000
_D = 128
_H = 64
_DE = 16
_L = 3
_G = 256

_NP = 10240
_B = 64
_EP = 323584
_PER_S = _EP // 16
_NBATCH = _PER_S // _B
_STRIPE = _NP // 16


def _edge_embed_body(ef_ref, w_ref, b_ref, e0_ref, e1_ref, e2_ref):
    y = jnp.dot(ef_ref[...], w_ref[...], preferred_element_type=jnp.float32)
    y = y + b_ref[...]
    for l, ref in enumerate((e0_ref, e1_ref, e2_ref)):
        ref[0] = y[:, l * _D:l * _D + _H]
        ref[1] = y[:, l * _D + _H:(l + 1) * _D]


def _edge_embed(ef, w, b):
    be = 2048
    eshape = jax.ShapeDtypeStruct((2, _EP, _H), jnp.float32)
    espec = pl.BlockSpec((2, be, _H), lambda i: (0, i, 0))
    return pl.pallas_call(
        _edge_embed_body,
        grid=(_EP // be,),
        in_specs=[
            pl.BlockSpec((be, _DE), lambda i: (i, 0)),
            pl.BlockSpec((_DE, _L * _D), lambda i: (0, 0)),
            pl.BlockSpec((1, _L * _D), lambda i: (0, 0)),
        ],
        out_specs=[espec, espec, espec],
        out_shape=[eshape, eshape, eshape],
        compiler_params=pltpu.CompilerParams(
            dimension_semantics=("parallel",)),
    )(ef, w, b)


def _split_outs(y, k_ref, qv_ref, xs_ref):
    k_ref[0] = y[:, 0:_H]
    k_ref[1] = y[:, _H:_D]
    qv_ref[0] = jnp.concatenate(
        [y[:, _D:_D + _H], y[:, 2 * _D:2 * _D + _H]], axis=1)
    qv_ref[1] = jnp.concatenate(
        [y[:, _D + _H:2 * _D], y[:, 2 * _D + _H:3 * _D]], axis=1)
    xs_ref[...] = y[:, 3 * _D:]


def _dense0_body(x_ref, w_ref, b_ref, k_ref, qv_ref, xs_ref):
    y = jnp.dot(x_ref[...], w_ref[...], preferred_element_type=jnp.float32)
    _split_outs(y + b_ref[...], k_ref, qv_ref, xs_ref)


def _dense_relu_body(p0_ref, p1_ref, xsin_ref, w_ref, b_ref,
                     k_ref, qv_ref, xs_ref):
    agg = jnp.concatenate([p0_ref[0], p1_ref[0]], axis=1)
    x = jnp.maximum(agg + xsin_ref[...], 0.0)
    y = jnp.dot(x, w_ref[...], preferred_element_type=jnp.float32)
    _split_outs(y + b_ref[...], k_ref, qv_ref, xs_ref)


def _dense_outs():
    return [
        jax.ShapeDtypeStruct((2, _NP, _H), jnp.float32),
        jax.ShapeDtypeStruct((2, _NP, 2 * _H), jnp.float32),
        jax.ShapeDtypeStruct((_NP, _D), jnp.float32),
    ]


def _dense_out_specs(bn):
    return [
        pl.BlockSpec((2, bn, _H), lambda i: (0, i, 0)),
        pl.BlockSpec((2, bn, 2 * _H), lambda i: (0, i, 0)),
        pl.BlockSpec((bn, _D), lambda i: (i, 0)),
    ]


def _dense0(x, w, b):
    bn = 512
    return pl.pallas_call(
        _dense0_body,
        grid=(_NP // bn,),
        in_specs=[
            pl.BlockSpec((bn, _D), lambda i: (i, 0)),
            pl.BlockSpec((_D, 4 * _D), lambda i: (0, 0)),
            pl.BlockSpec((1, 4 * _D), lambda i: (0, 0)),
        ],
        out_specs=_dense_out_specs(bn),
        out_shape=_dense_outs(),
        compiler_params=pltpu.CompilerParams(
            dimension_semantics=("parallel",)),
    )(x, w, b)


def _dense_relu(parts, xs_in, w, b):
    bn = 512
    return pl.pallas_call(
        _dense_relu_body,
        grid=(_NP // bn,),
        in_specs=[
            pl.BlockSpec((1, bn, _H), lambda i: (0, i, 0)),
            pl.BlockSpec((1, bn, _H), lambda i: (1, i, 0)),
            pl.BlockSpec((bn, _D), lambda i: (i, 0)),
            pl.BlockSpec((_D, 4 * _D), lambda i: (0, 0)),
            pl.BlockSpec((1, 4 * _D), lambda i: (0, 0)),
        ],
        out_specs=_dense_out_specs(bn),
        out_shape=_dense_outs(),
        compiler_params=pltpu.CompilerParams(
            dimension_semantics=("parallel",)),
    )(parts, parts, xs_in, w, b)


def _sc_edge(k, qv, e, src, dst, zeros):
    mesh = plsc.VectorSubcoreMesh(core_axis_name="c", subcore_axis_name="s")

    @functools.partial(
        pl.kernel,
        out_type=jax.ShapeDtypeStruct((2, _NP, _H), jnp.float32),
        mesh=mesh,
        scratch_types=[
            pltpu.VMEM((_NBATCH, _B), jnp.int32),
            pltpu.VMEM((_NBATCH, _B), jnp.int32),
            [pltpu.VMEM((_B, _H), jnp.float32) for _ in range(2)],
            [pltpu.VMEM((_B, 2 * _H), jnp.float32) for _ in range(2)],
            [pltpu.VMEM((_B, _H), jnp.float32) for _ in range(2)],
            [pltpu.VMEM((_B, _H), jnp.float32) for _ in range(2)],
            [pltpu.SemaphoreType.DMA for _ in range(2)],
            [pltpu.SemaphoreType.DMA for _ in range(2)],
            pltpu.VMEM_SHARED((_NP, _H), jnp.float32),
        ],
        compiler_params=pltpu.CompilerParams(use_tc_tiling_on_sc=False),
    )
    def kern(k_hbm, qv_hbm, e_hbm, src_hbm, dst_hbm, z_hbm, out_hbm,
             src_v, dst_v, kb, qvb, eb, mb, sem, ssem, agg):
        c = lax.axis_index("c")
        s = lax.axis_index("s")
        pltpu.sync_copy(z_hbm.at[pl.ds(s * _STRIPE, _STRIPE)],
                        agg.at[pl.ds(s * _STRIPE, _STRIPE)])
        pltpu.sync_copy(src_hbm.at[s], src_v)
        pltpu.sync_copy(dst_hbm.at[s], dst_v)
        plsc.subcore_barrier()
        base0 = s * _PER_S

        def gather_descr(bi, slot):
            base = base0 + bi * _B
            return (
                pltpu.make_async_copy(k_hbm.at[c].at[dst_v.at[bi]], kb[slot],
                                      sem[slot]),
                pltpu.make_async_copy(qv_hbm.at[c].at[src_v.at[bi]],
                                      qvb[slot], sem[slot]),
                pltpu.make_async_copy(e_hbm.at[c, pl.ds(base, _B)], eb[slot],
                                      sem[slot]),
            )

        def issue(bi, slot):
            for d in gather_descr(bi, slot):
                d.start()

        def scatter_wait(bi, slot):
            pltpu.make_async_copy(mb[slot], agg.at[dst_v.at[bi]],
                                  ssem[slot]).wait()

        def finish(bi, slot):
            for d in gather_descr(bi, slot):
                d.wait()

            @pl.when(bi >= 2)
            def _():
                scatter_wait(bi - 2, slot)

            @plsc.parallel_loop(0, _B, unroll=2)
            def _row(i):
                for j in range(_H // 16):
                    sl = pl.ds(j * 16, 16)
                    z = kb[slot][i, sl] + qvb[slot][i, sl] + eb[slot][i, sl]
                    vv = qvb[slot][i, pl.ds(_H + j * 16, 16)]
                    mb[slot][i, sl] = vv / (1.0 + jnp.exp(-z))

            pltpu.async_copy(mb[slot], agg.at[dst_v.at[bi]], ssem[slot],
                             add=True)

        issue(0, 0)

        @pl.loop(0, _NBATCH, step=2)
        def _batch(bi):
            issue(bi + 1, 1)
            finish(bi, 0)

            @pl.when(bi + 2 < _NBATCH)
            def _():
                issue(bi + 2, 0)

            finish(bi + 1, 1)

        scatter_wait(_NBATCH - 2, 0)
        scatter_wait(_NBATCH - 1, 1)
        plsc.subcore_barrier()
        pltpu.sync_copy(agg.at[pl.ds(s * _STRIPE, _STRIPE)],
                        out_hbm.at[c, pl.ds(s * _STRIPE, _STRIPE)])

    return kern(k, qv, e, src, dst, zeros)


def _head_body(p0_ref, p1_ref, xs_ref, b3_ref, wg_ref, bg_ref,
               lng_ref, lnb_ref, lgg_ref, lgb_ref,
               loc_ref, glob_ref, gsum_ref, gcnt_ref):
    i = pl.program_id(0)
    nb = pl.num_programs(0)
    agg = jnp.concatenate([p0_ref[0], p1_ref[0]], axis=1)
    x = agg + xs_ref[...]

    m = jnp.mean(x, axis=-1, keepdims=True)
    v = jnp.mean((x - m) ** 2, axis=-1, keepdims=True)
    loc_ref[...] = ((x - m) * lax.rsqrt(v + 1e-5) * lng_ref[...]
                    + lnb_ref[...])

    @pl.when(i == 0)
    def _():
        gsum_ref[...] = jnp.zeros_like(gsum_ref)
        gcnt_ref[...] = jnp.zeros_like(gcnt_ref)

    seg = b3_ref[0]
    segb = jnp.broadcast_to(seg, (_G, seg.shape[1]))
    gids = lax.broadcasted_iota(jnp.int32, (_G, seg.shape[1]), 0)
    mt = (segb == gids).astype(jnp.float32)
    gsum_ref[...] += jnp.dot(mt, x, preferred_element_type=jnp.float32)
    gcnt_ref[...] += jnp.dot(mt, jnp.ones_like(x),
                             preferred_element_type=jnp.float32)

    @pl.when(i == nb - 1)
    def _():
        gmean = gsum_ref[...] / jnp.maximum(gcnt_ref[...], 1.0)
        gg = jnp.dot(gmean, wg_ref[...],
                     preferred_element_type=jnp.float32) + bg_ref[...]
        m2 = jnp.mean(gg, axis=-1, keepdims=True)
        v2 = jnp.mean((gg - m2) ** 2, axis=-1, keepdims=True)
        glob_ref[...] = ((gg - m2) * lax.rsqrt(v2 + 1e-5) * lgg_ref[...]
                         + lgb_ref[...])


def _head(parts, xs_in, batch3, wg, bg, lng, lnb, lgg, lgb):
    bh = 400
    return pl.pallas_call(
        _head_body,
        grid=(_N // bh,),
        in_specs=[
            pl.BlockSpec((1, bh, _H), lambda i: (0, i, 0)),
            pl.BlockSpec((1, bh, _H), lambda i: (1, i, 0)),
            pl.BlockSpec((bh, _D), lambda i: (i, 0)),
            pl.BlockSpec((1, 1, bh), lambda i: (i, 0, 0)),
            pl.BlockSpec((_D, _D), lambda i: (0, 0)),
            pl.BlockSpec((1, _D), lambda i: (0, 0)),
            pl.BlockSpec((1, _D), lambda i: (0, 0)),
            pl.BlockSpec((1, _D), lambda i: (0, 0)),
            pl.BlockSpec((1, _D), lambda i: (0, 0)),
            pl.BlockSpec((1, _D), lambda i: (0, 0)),
        ],
        out_specs=[
            pl.BlockSpec((bh, _D), lambda i: (i, 0)),
            pl.BlockSpec((_G, _D), lambda i: (0, 0)),
        ],
        out_shape=[
            jax.ShapeDtypeStruct((_N, _D), jnp.float32),
            jax.ShapeDtypeStruct((_G, _D), jnp.float32),
        ],
        scratch_shapes=[
            pltpu.VMEM((_G, _D), jnp.float32),
            pltpu.VMEM((_G, _D), jnp.float32),
        ],
        compiler_params=pltpu.CompilerParams(
            dimension_semantics=("arbitrary",)),
    )(parts, parts, xs_in, batch3, wg, bg, lng, lnb, lgg, lgb)


def kernel(node_feature, edge_index, edge_feature, batch, Wk, bk, Wq, bq,
           Wv, bv, We, be, Ws, bconv, Wg, bg, ln_node_g, ln_node_b,
           ln_graph_g, ln_graph_b):
    f32 = jnp.float32
    x0 = jnp.pad(node_feature, ((0, _NP - _N), (0, 0)))
    src = jnp.pad(edge_index[0], (0, _EP - _E)).reshape(16, _NBATCH, _B)
    dst = jnp.pad(edge_index[1], (0, _EP - _E),
                  constant_values=_N).reshape(16, _NBATCH, _B)
    ef = jnp.pad(edge_feature, ((0, _EP - _E), (0, 0)))
    batch3 = batch.reshape(_N // 400, 1, 400)
    zeros = jnp.zeros((_NP, _H), f32)

    wall = jnp.concatenate([Wk, Wq, Wv, Ws], axis=2)
    ball = jnp.concatenate(
        [bk, bq, bv, bconv], axis=1).reshape(_L, 1, 4 * _D)
    we_all = jnp.transpose(We, (1, 0, 2)).reshape(_DE, _L * _D)
    be_all = be.reshape(1, _L * _D)

    e0, e1, e2 = _edge_embed(ef, we_all, be_all)

    k, qv, xs = _dense0(x0, wall[0], ball[0])
    parts = _sc_edge(k, qv, e0, src, dst, zeros)
    k, qv, xs = _dense_relu(parts, xs, wall[1], ball[1])
    parts = _sc_edge(k, qv, e1, src, dst, zeros)
    k, qv, xs = _dense_relu(parts, xs, wall[2], ball[2])
    parts = _sc_edge(k, qv, e2, src, dst, zeros)

    loc, glob = _head(parts, xs, batch3,
                      Wg, bg.reshape(1, _D),
                      ln_node_g.reshape(1, _D), ln_node_b.reshape(1, _D),
                      ln_graph_g.reshape(1, _D), ln_graph_b.reshape(1, _D))
    return loc, glob
